# Initial kernel scaffold; baseline (speedup 1.0000x reference)
import jax
import jax.numpy as jnp
from jax.experimental import pallas as pl


def kernel(x, table, W1, b1, W2, b2):
    raise NotImplementedError("write your pallas kernel here")


# speedup vs baseline: 5.5437x; 5.5437x over previous
import jax
import jax.numpy as jnp
from jax import lax
from jax.experimental import pallas as pl
from jax.experimental.pallas import tpu as pltpu
from jax.experimental.pallas import tpu_sc as plsc

VOCAB = 100000
DIM = 64
BATCH = 4096
HIST = 50
FEAT = 16

NC = 2
NS = 16
NW = NC * NS
BPW = BATCH // NW
GS = 2
NG = BPW // GS
NBUF = 8
NT = NG // NBUF

_MM_BLK = 1000
_FC2_BLK = 512


def _fc1_project(table, W1):
    def body(t_ref, w_ref, o_ref):
        o_ref[...] = jnp.dot(t_ref[...], w_ref[...],
                             preferred_element_type=jnp.float32)

    return pl.pallas_call(
        body,
        grid=(VOCAB // _MM_BLK,),
        in_specs=[
            pl.BlockSpec((_MM_BLK, DIM), lambda i: (i, 0)),
            pl.BlockSpec((DIM, FEAT), lambda i: (0, 0)),
        ],
        out_specs=pl.BlockSpec((_MM_BLK, FEAT), lambda i: (i, 0)),
        out_shape=jax.ShapeDtypeStruct((VOCAB, FEAT), jnp.float32),
    )(table, W1)


def _sc_body(g_hbm, x_hbm, b1_hbm, out_hbm, idx_v, rows_v, out_v, b1_v,
             *sems):
    wid = lax.axis_index("s") * NC + lax.axis_index("c")

    pltpu.sync_copy(x_hbm.at[wid], idx_v)
    pltpu.sync_copy(b1_hbm, b1_v)
    b1 = b1_v[...]
    inv_h = jnp.float32(1.0 / HIST)

    for b in range(NBUF):
        pltpu.async_copy(g_hbm.at[idx_v.at[b]], rows_v.at[b], sems[b])

    def outer(t, carry):
        for b in range(NBUF):
            g = t * NBUF + b
            pltpu.make_async_copy(g_hbm.at[idx_v.at[g]], rows_v.at[b],
                                  sems[b]).wait()
            for e in range(GS):
                base_r = e * HIST
                a0 = rows_v[b, base_r, :]
                a1 = rows_v[b, base_r + 1, :]
                for j in range(2, HIST, 2):
                    a0 = a0 + rows_v[b, base_r + j, :]
                    a1 = a1 + rows_v[b, base_r + j + 1, :]
                acc = a0 + a1
                out_v[g * GS + e] = jnp.maximum(acc * inv_h + b1, 0.0)
            @pl.when(g + NBUF < NG)
            def _():
                pltpu.async_copy(g_hbm.at[idx_v.at[g + NBUF]], rows_v.at[b],
                                 sems[b])
        return carry

    lax.fori_loop(0, NT, outer, 0)
    pltpu.sync_copy(out_v, out_hbm.at[pl.ds(wid * BPW, BPW)])


def _sc_pool_hidden(g, x_grouped, b1):
    mesh = plsc.VectorSubcoreMesh(core_axis_name="c", subcore_axis_name="s")
    kfn = pl.kernel(
        _sc_body,
        out_type=jax.ShapeDtypeStruct((BATCH, FEAT), jnp.float32),
        mesh=mesh,
        scratch_types=[
            pltpu.VMEM((NG, GS * HIST), jnp.int32),
            pltpu.VMEM((NBUF, GS * HIST, FEAT), jnp.float32),
            pltpu.VMEM((BPW, FEAT), jnp.float32),
            pltpu.VMEM((FEAT,), jnp.float32),
        ] + [pltpu.SemaphoreType.DMA] * NBUF,
        compiler_params=pltpu.CompilerParams(use_tc_tiling_on_sc=False),
    )
    return kfn(g, x_grouped, b1)


def _fc2(h, W2, b2):
    def body(h_ref, w_ref, b_ref, o_ref):
        o_ref[...] = jnp.dot(h_ref[...], w_ref[...],
                             preferred_element_type=jnp.float32) + b_ref[...]

    return pl.pallas_call(
        body,
        grid=(BATCH // _FC2_BLK,),
        in_specs=[
            pl.BlockSpec((_FC2_BLK, FEAT), lambda i: (i, 0)),
            pl.BlockSpec((FEAT, 1), lambda i: (0, 0)),
            pl.BlockSpec((1, 1), lambda i: (0, 0)),
        ],
        out_specs=pl.BlockSpec((_FC2_BLK, 1), lambda i: (i, 0)),
        out_shape=jax.ShapeDtypeStruct((BATCH, 1), jnp.float32),
    )(h, W2, b2.reshape(1, 1))


def kernel(x, table, W1, b1, W2, b2):
    g = _fc1_project(table, W1)
    x_grouped = x.astype(jnp.int32).reshape(NW, NG, GS * HIST)
    h = _sc_pool_hidden(g, x_grouped, b1)
    return _fc2(h, W2, b2)


# speedup vs baseline: 7.6664x; 1.3829x over previous
import jax
import jax.numpy as jnp
from jax import lax
from jax.experimental import pallas as pl
from jax.experimental.pallas import tpu as pltpu
from jax.experimental.pallas import tpu_sc as plsc

VOCAB = 100000
DIM = 64
BATCH = 4096
HIST = 50
FEAT = 16

NC = 2
NS = 16
NW = NC * NS
BPW = BATCH // NW
GS = 2
NG = BPW // GS
NBUF = 4
NT = NG // NBUF

_MLP_BLK = 512


def _sc_body(tab_hbm, x_hbm, out_hbm, idx_v, rows_v, out_v, *sems):
    wid = lax.axis_index("s") * NC + lax.axis_index("c")

    pltpu.sync_copy(x_hbm.at[wid], idx_v)
    inv_h = jnp.float32(1.0 / HIST)

    for b in range(NBUF):
        pltpu.async_copy(tab_hbm.at[idx_v.at[b]], rows_v.at[b], sems[b])

    def outer(t, carry):
        for b in range(NBUF):
            g = t * NBUF + b
            pltpu.make_async_copy(tab_hbm.at[idx_v.at[g]], rows_v.at[b],
                                  sems[b]).wait()
            for e in range(GS):
                base_r = e * HIST
                for c in range(DIM // 16):
                    lo, hi = c * 16, (c + 1) * 16
                    a0 = rows_v[b, base_r, lo:hi]
                    a1 = rows_v[b, base_r + 1, lo:hi]
                    for j in range(2, HIST, 2):
                        a0 = a0 + rows_v[b, base_r + j, lo:hi]
                        a1 = a1 + rows_v[b, base_r + j + 1, lo:hi]
                    out_v[g * GS + e, lo:hi] = (a0 + a1) * inv_h
            @pl.when(g + NBUF < NG)
            def _():
                pltpu.async_copy(tab_hbm.at[idx_v.at[g + NBUF]], rows_v.at[b],
                                 sems[b])
        return carry

    lax.fori_loop(0, NT, outer, 0)
    pltpu.sync_copy(out_v, out_hbm.at[pl.ds(wid * BPW, BPW)])


def _sc_pool(table, x_grouped):
    mesh = plsc.VectorSubcoreMesh(core_axis_name="c", subcore_axis_name="s")
    kfn = pl.kernel(
        _sc_body,
        out_type=jax.ShapeDtypeStruct((BATCH, DIM), jnp.float32),
        mesh=mesh,
        scratch_types=[
            pltpu.VMEM((NG, GS * HIST), jnp.int32),
            pltpu.VMEM((NBUF, GS * HIST, DIM), jnp.float32),
            pltpu.VMEM((BPW, DIM), jnp.float32),
        ] + [pltpu.SemaphoreType.DMA] * NBUF,
        compiler_params=pltpu.CompilerParams(use_tc_tiling_on_sc=False),
    )
    return kfn(table, x_grouped)


def _mlp(mean, W1, b1, W2, b2):
    def body(m_ref, w1_ref, b1_ref, w2_ref, b2_ref, o_ref):
        h = jnp.dot(m_ref[...], w1_ref[...],
                    preferred_element_type=jnp.float32) + b1_ref[...]
        h = jnp.maximum(h, 0.0)
        o_ref[...] = jnp.dot(h, w2_ref[...],
                             preferred_element_type=jnp.float32) + b2_ref[...]

    return pl.pallas_call(
        body,
        grid=(BATCH // _MLP_BLK,),
        in_specs=[
            pl.BlockSpec((_MLP_BLK, DIM), lambda i: (i, 0)),
            pl.BlockSpec((DIM, FEAT), lambda i: (0, 0)),
            pl.BlockSpec((1, FEAT), lambda i: (0, 0)),
            pl.BlockSpec((FEAT, 1), lambda i: (0, 0)),
            pl.BlockSpec((1, 1), lambda i: (0, 0)),
        ],
        out_specs=pl.BlockSpec((_MLP_BLK, 1), lambda i: (i, 0)),
        out_shape=jax.ShapeDtypeStruct((BATCH, 1), jnp.float32),
    )(mean, W1, b1.reshape(1, FEAT), W2, b2.reshape(1, 1))


def kernel(x, table, W1, b1, W2, b2):
    x_grouped = x.astype(jnp.int32).reshape(NW, NG, GS * HIST)
    mean = _sc_pool(table, x_grouped)
    return _mlp(mean, W1, b1, W2, b2)
